# Initial kernel scaffold; baseline (speedup 1.0000x reference)
import jax
import jax.numpy as jnp
from jax.experimental import pallas as pl


def kernel(commands, args, cmd_table, arg_table, W, b):
    raise NotImplementedError("write your pallas kernel here")


# speedup vs baseline: 5.4323x; 5.4323x over previous
import functools

import jax
import jax.numpy as jnp
from jax import lax
from jax.experimental import pallas as pl
from jax.experimental.pallas import tpu as pltpu
from jax.experimental.pallas import tpu_sc as plsc

N, S, ARG_NUM = 1024, 64, 16
B = N * S
D = 128
E = 64
ROW_STRIDE = 264
CMD_BASE = ARG_NUM * ROW_STRIDE
TBL_ROWS = CMD_BASE + 8
NC, NS = 2, 16
NW = NC * NS
CHUNK = 128
TOK_PER_W = B // NW
NCHUNK = TOK_PER_W // CHUNK

_OFFS = [CMD_BASE] + [k * ROW_STRIDE + 1 for k in range(ARG_NUM)]


def _fold_body(cmd_ref, arg_ref, w_ref, b_ref, out_ref):
    a = arg_ref[...]
    for k in range(ARG_NUM):
        wk = w_ref[pl.ds(k * E, E), :]
        out_ref[pl.ds(k * ROW_STRIDE, ROW_STRIDE), :] = jnp.dot(
            a, wk, preferred_element_type=jnp.float32)
    out_ref[pl.ds(CMD_BASE, 8), :] = cmd_ref[...] + b_ref[...]


def _fold(cmd_pad, arg_pad, W, b2):
    return pl.pallas_call(
        _fold_body,
        out_shape=jax.ShapeDtypeStruct((TBL_ROWS, D), jnp.float32),
    )(cmd_pad, arg_pad, W, b2)


def _sc_body(table_hbm, catT_hbm, out_hbm, raw_v, idx_v, acc_v, sem):
    wid = lax.axis_index("s") * NC + lax.axis_index("c")

    def chunk_body(c, carry):
        base = wid * TOK_PER_W + c * CHUNK
        pltpu.sync_copy(catT_hbm.at[:, pl.ds(base, CHUNK)], raw_v)
        for k in range(17):
            off = _OFFS[k]
            for g in range(CHUNK // 16):
                sl = pl.ds(g * 16, 16)
                idx_v[k, sl] = raw_v[k, sl] + off
        pltpu.async_copy(table_hbm.at[idx_v.at[0]], acc_v, sem).wait()
        for k in range(1, 17):
            pltpu.async_copy(table_hbm.at[idx_v.at[k]], acc_v, sem, add=True)
        for k in range(1, 17):
            pltpu.make_async_copy(table_hbm.at[idx_v.at[0]], acc_v, sem).wait()
        pltpu.sync_copy(acc_v, out_hbm.at[pl.ds(base, CHUNK), :])
        return carry

    lax.fori_loop(0, NCHUNK, chunk_body, 0)


_sc_lookup = functools.partial(
    pl.kernel,
    out_type=jax.ShapeDtypeStruct((B, D), jnp.float32),
    mesh=plsc.VectorSubcoreMesh(core_axis_name="c", subcore_axis_name="s"),
    scratch_types=[
        pltpu.VMEM((17, CHUNK), jnp.int32),
        pltpu.VMEM((17, CHUNK), jnp.int32),
        pltpu.VMEM((CHUNK, D), jnp.float32),
        pltpu.SemaphoreType.DMA,
    ],
)(_sc_body)


def kernel(commands, args, cmd_table, arg_table, W, b):
    cmd_pad = jnp.zeros((8, D), jnp.float32).at[:6].set(cmd_table)
    arg_pad = jnp.zeros((ROW_STRIDE, E), jnp.float32).at[:257].set(arg_table)
    flat_table = _fold(cmd_pad, arg_pad, W, b.reshape(1, D))
    catT = jnp.concatenate(
        [commands.reshape(1, B), args.reshape(B, ARG_NUM).T], axis=0)
    out = _sc_lookup(flat_table, catT)
    return out.reshape(N, S, D)


# speedup vs baseline: 5.5288x; 1.0178x over previous
import functools

import jax
import jax.numpy as jnp
from jax import lax
from jax.experimental import pallas as pl
from jax.experimental.pallas import tpu as pltpu
from jax.experimental.pallas import tpu_sc as plsc

N, S, ARG_NUM = 1024, 64, 16
B = N * S
D = 128
E = 64
ROW_STRIDE = 264
CMD_BASE = ARG_NUM * ROW_STRIDE
TBL_ROWS = CMD_BASE + 8
NC, NS = 2, 16
NW = NC * NS
CHUNK = 256
G = CHUNK // 128
NCHUNK = (B // NW) // CHUNK
TOK_PER_W = B // NW

_OFFS = [CMD_BASE] + [k * ROW_STRIDE + 1 for k in range(ARG_NUM)]


def _fold_body(cmd_ref, arg_ref, w_ref, b_ref, out_ref):
    a = arg_ref[...]
    for k in range(ARG_NUM):
        wk = w_ref[pl.ds(k * E, E), :]
        out_ref[pl.ds(k * ROW_STRIDE, ROW_STRIDE), :] = jnp.dot(
            a, wk, preferred_element_type=jnp.float32)
    out_ref[pl.ds(CMD_BASE, 8), :] = cmd_ref[...] + b_ref[...]


def _fold(cmd_pad, arg_pad, W, b2):
    return pl.pallas_call(
        _fold_body,
        out_shape=jax.ShapeDtypeStruct((TBL_ROWS, D), jnp.float32),
    )(cmd_pad, arg_pad, W, b2)


def _sc_body(table, catT, out, raw, idx, acc,
             sg0, sg1, sr0, sr1, so0, so1):
    sg = [sg0, sg1]
    sr = [sr0, sr1]
    so = [so0, so1]
    wid = lax.axis_index("s") * NC + lax.axis_index("c")
    base0 = wid * TOK_PER_W

    def wait_raw(b, base):
        pltpu.make_async_copy(catT.at[:, pl.ds(base, CHUNK)],
                              raw.at[b], sr[b]).wait()

    def fire_raw(b, base):
        pltpu.async_copy(catT.at[:, pl.ds(base, CHUNK)], raw.at[b], sr[b])

    def adjust(b):
        def body(g8, carry):
            for k in range(17):
                for j in range(G):
                    v = raw[b, k, pl.ds(j * 128 + g8 * 16, 16)] + _OFFS[k]
                    idx[b, G * k + j, pl.ds(g8 * 16, 16)] = v
            return carry
        lax.fori_loop(0, 8, body, 0)

    def zero_acc(b):
        z = jnp.zeros((16,), jnp.float32)

        def body(r, carry):
            for j in range(D // 16):
                acc[b, r, pl.ds(j * 16, 16)] = z
            return carry
        lax.fori_loop(0, CHUNK, body, 0)

    def fire_gathers(b):
        for k in range(17):
            for j in range(G):
                pltpu.async_copy(table.at[idx.at[b, G * k + j]],
                                 acc.at[b, pl.ds(j * 128, 128), :],
                                 sg[b], add=True)

    def drain_gathers(b):
        for _ in range(17 * G):
            pltpu.make_async_copy(table.at[idx.at[b, 0]],
                                  acc.at[b, pl.ds(0, 128), :], sg[b]).wait()

    def fire_out(b, base):
        pltpu.async_copy(acc.at[b], out.at[pl.ds(base, CHUNK), :], so[b])

    def wait_out(b):
        pltpu.make_async_copy(acc.at[b], out.at[pl.ds(0, CHUNK), :],
                              so[b]).wait()

    def chunk(b, base, first=False, second=False, fire_next=True):
        wait_raw(b, base)
        adjust(b)
        if fire_next:
            fire_raw(1 - b, base + CHUNK)
        if not (first or second):
            wait_out(b)
        zero_acc(b)
        fire_gathers(b)
        if not first:
            drain_gathers(1 - b)
            fire_out(1 - b, base - CHUNK)

    fire_raw(0, base0)
    chunk(0, base0, first=True)
    chunk(1, base0 + CHUNK, second=True)

    def pair(p, carry):
        base = base0 + 2 * p * CHUNK
        chunk(0, base)
        chunk(1, base + CHUNK)
        return carry
    lax.fori_loop(1, NCHUNK // 2 - 1, pair, 0)

    baseL = base0 + (NCHUNK - 2) * CHUNK
    chunk(0, baseL)
    chunk(1, baseL + CHUNK, fire_next=False)
    drain_gathers(1)
    fire_out(1, baseL + CHUNK)
    wait_out(0)
    wait_out(1)


_sc_lookup = functools.partial(
    pl.kernel,
    out_type=jax.ShapeDtypeStruct((B, D), jnp.float32),
    mesh=plsc.VectorSubcoreMesh(core_axis_name="c", subcore_axis_name="s"),
    scratch_types=[
        pltpu.VMEM((2, 17, CHUNK), jnp.int32),
        pltpu.VMEM((2, 17 * G, 128), jnp.int32),
        pltpu.VMEM((2, CHUNK, D), jnp.float32),
        pltpu.SemaphoreType.DMA,
        pltpu.SemaphoreType.DMA,
        pltpu.SemaphoreType.DMA,
        pltpu.SemaphoreType.DMA,
        pltpu.SemaphoreType.DMA,
        pltpu.SemaphoreType.DMA,
    ],
)(_sc_body)


def kernel(commands, args, cmd_table, arg_table, W, b):
    cmd_pad = jnp.zeros((8, D), jnp.float32).at[:6].set(cmd_table)
    arg_pad = jnp.zeros((ROW_STRIDE, E), jnp.float32).at[:257].set(arg_table)
    flat_table = _fold(cmd_pad, arg_pad, W, b.reshape(1, D))
    catT = jnp.concatenate(
        [commands.reshape(1, B), args.reshape(B, ARG_NUM).T], axis=0)
    out = _sc_lookup(flat_table, catT)
    return out.reshape(N, S, D)


# speedup vs baseline: 16.9529x; 3.0663x over previous
import functools

import jax
import jax.numpy as jnp
from jax import lax
from jax.experimental import pallas as pl
from jax.experimental.pallas import tpu as pltpu
from jax.experimental.pallas import tpu_sc as plsc

N, S, ARG_NUM = 1024, 64, 16
B = N * S
D = 128
E = 64
ROW_STRIDE = 264
CMD_BASE = ARG_NUM * ROW_STRIDE
TBL_ROWS = CMD_BASE + 8
NC, NS = 2, 16
NW = NC * NS
CHUNK = 256
G = CHUNK // 128
NCHUNK = (B // NW) // CHUNK
TOK_PER_W = B // NW

_OFFS = [CMD_BASE] + [k * ROW_STRIDE + 1 for k in range(ARG_NUM)]


def _fold_body(cmd_ref, arg_ref, w_ref, b_ref, out_ref):
    a = arg_ref[...]
    for k in range(ARG_NUM):
        wk = w_ref[pl.ds(k * E, E), :]
        out_ref[pl.ds(k * ROW_STRIDE, ROW_STRIDE), :] = jnp.dot(
            a, wk, preferred_element_type=jnp.float32)
    out_ref[pl.ds(CMD_BASE, 8), :] = cmd_ref[...] + b_ref[...]


def _fold(cmd_pad, arg_pad, W, b2):
    return pl.pallas_call(
        _fold_body,
        out_shape=jax.ShapeDtypeStruct((TBL_ROWS, D), jnp.float32),
    )(cmd_pad, arg_pad, W, b2)


def _sc_body(table_hbm, catT, out, raw, idx, acc, table,
             sg0, sg1, sr0, sr1, so0, so1):
    sg = [sg0, sg1]
    sr = [sr0, sr1]
    so = [so0, so1]
    sid = lax.axis_index("s")
    wid = sid * NC + lax.axis_index("c")
    base0 = wid * TOK_PER_W

    @pl.when(sid == 0)
    def _stage():
        pltpu.sync_copy(table_hbm, table)
    plsc.subcore_barrier()

    def wait_raw(b, base):
        pltpu.make_async_copy(catT.at[:, pl.ds(base, CHUNK)],
                              raw.at[b], sr[b]).wait()

    def fire_raw(b, base):
        pltpu.async_copy(catT.at[:, pl.ds(base, CHUNK)], raw.at[b], sr[b])

    def adjust(b):
        def body(g8, carry):
            for k in range(17):
                for j in range(G):
                    v = raw[b, k, pl.ds(j * 128 + g8 * 16, 16)] + _OFFS[k]
                    idx[b, G * k + j, pl.ds(g8 * 16, 16)] = v
            return carry
        lax.fori_loop(0, 8, body, 0)

    def zero_acc(b):
        z = jnp.zeros((16,), jnp.float32)

        def body(r, carry):
            for j in range(D // 16):
                acc[b, r, pl.ds(j * 16, 16)] = z
            return carry
        lax.fori_loop(0, CHUNK, body, 0)

    def fire_gathers(b):
        for k in range(17):
            for j in range(G):
                pltpu.async_copy(table.at[idx.at[b, G * k + j]],
                                 acc.at[b, pl.ds(j * 128, 128), :],
                                 sg[b], add=True)

    def drain_gathers(b):
        for _ in range(17 * G):
            pltpu.make_async_copy(table.at[idx.at[b, 0]],
                                  acc.at[b, pl.ds(0, 128), :], sg[b]).wait()

    def fire_out(b, base):
        pltpu.async_copy(acc.at[b], out.at[pl.ds(base, CHUNK), :], so[b])

    def wait_out(b):
        pltpu.make_async_copy(acc.at[b], out.at[pl.ds(0, CHUNK), :],
                              so[b]).wait()

    def chunk(b, base, first=False, second=False, fire_next=True):
        wait_raw(b, base)
        adjust(b)
        if fire_next:
            fire_raw(1 - b, base + CHUNK)
        if not (first or second):
            wait_out(b)
        zero_acc(b)
        fire_gathers(b)
        if not first:
            drain_gathers(1 - b)
            fire_out(1 - b, base - CHUNK)

    fire_raw(0, base0)
    chunk(0, base0, first=True)
    chunk(1, base0 + CHUNK, second=True)

    def pair(p, carry):
        base = base0 + 2 * p * CHUNK
        chunk(0, base)
        chunk(1, base + CHUNK)
        return carry
    lax.fori_loop(1, NCHUNK // 2 - 1, pair, 0)

    baseL = base0 + (NCHUNK - 2) * CHUNK
    chunk(0, baseL)
    chunk(1, baseL + CHUNK, fire_next=False)
    drain_gathers(1)
    fire_out(1, baseL + CHUNK)
    wait_out(0)
    wait_out(1)


_sc_lookup = functools.partial(
    pl.kernel,
    out_type=jax.ShapeDtypeStruct((B, D), jnp.float32),
    mesh=plsc.VectorSubcoreMesh(core_axis_name="c", subcore_axis_name="s"),
    scratch_types=[
        pltpu.VMEM((2, 17, CHUNK), jnp.int32),
        pltpu.VMEM((2, 17 * G, 128), jnp.int32),
        pltpu.VMEM((2, CHUNK, D), jnp.float32),
        pltpu.VMEM_SHARED((TBL_ROWS, D), jnp.float32),
        pltpu.SemaphoreType.DMA,
        pltpu.SemaphoreType.DMA,
        pltpu.SemaphoreType.DMA,
        pltpu.SemaphoreType.DMA,
        pltpu.SemaphoreType.DMA,
        pltpu.SemaphoreType.DMA,
    ],
)(_sc_body)


def kernel(commands, args, cmd_table, arg_table, W, b):
    cmd_pad = jnp.zeros((8, D), jnp.float32).at[:6].set(cmd_table)
    arg_pad = jnp.zeros((ROW_STRIDE, E), jnp.float32).at[:257].set(arg_table)
    flat_table = _fold(cmd_pad, arg_pad, W, b.reshape(1, D))
    catT = jnp.concatenate(
        [commands.reshape(1, B), args.reshape(B, ARG_NUM).T], axis=0)
    out = _sc_lookup(flat_table, catT)
    return out.reshape(N, S, D)


# speedup vs baseline: 17.1512x; 1.0117x over previous
import functools

import jax
import jax.numpy as jnp
from jax import lax
from jax.experimental import pallas as pl
from jax.experimental.pallas import tpu as pltpu
from jax.experimental.pallas import tpu_sc as plsc

N, S, ARG_NUM = 1024, 64, 16
B = N * S
D = 128
E = 64
ROW_STRIDE = 264
CMD_BASE = ARG_NUM * ROW_STRIDE
TBL_ROWS = CMD_BASE + 8
NC, NS = 2, 16
NW = NC * NS
CHUNK = 256
G = CHUNK // 128
NCHUNK = (B // NW) // CHUNK
TOK_PER_W = B // NW


def _fold_body(cmd_ref, arg_ref, w_ref, b_ref, out_ref):
    a = arg_ref[...]
    for k in range(ARG_NUM):
        wk = w_ref[pl.ds(k * E, E), :]
        out_ref[pl.ds(k * ROW_STRIDE, 257), :] = jnp.dot(
            a, wk, preferred_element_type=jnp.float32)
    out_ref[pl.ds(CMD_BASE, 6), :] = cmd_ref[...] + b_ref[...]


def _fold(cmd_table, arg_table, W, b2):
    return pl.pallas_call(
        _fold_body,
        out_shape=jax.ShapeDtypeStruct((TBL_ROWS, D), jnp.float32),
    )(cmd_table, arg_table, W, b2)


def _sc_body(table_hbm, cmdf, argsT, out, rawc, rawa, idx, acc, table,
             sg0, sg1, sr0, sr1, sra0, sra1, so0, so1):
    sg = [sg0, sg1]
    sr = [sr0, sr1]
    sra = [sra0, sra1]
    so = [so0, so1]
    sid = lax.axis_index("s")
    wid = sid * NC + lax.axis_index("c")
    base0 = wid * TOK_PER_W

    @pl.when(sid == 0)
    def _stage():
        pltpu.sync_copy(table_hbm, table)
    plsc.subcore_barrier()

    def wait_raw(b, base):
        pltpu.make_async_copy(cmdf.at[pl.ds(base, CHUNK)],
                              rawc.at[b], sr[b]).wait()
        pltpu.make_async_copy(argsT.at[:, pl.ds(base, CHUNK)],
                              rawa.at[b], sra[b]).wait()

    def fire_raw(b, base):
        pltpu.async_copy(cmdf.at[pl.ds(base, CHUNK)], rawc.at[b], sr[b])
        pltpu.async_copy(argsT.at[:, pl.ds(base, CHUNK)], rawa.at[b], sra[b])

    def adjust(b):
        def body(g8, carry):
            for j in range(G):
                t0 = j * 128 + g8 * 16
                idx[b, j, pl.ds(g8 * 16, 16)] = (
                    rawc[b, pl.ds(t0, 16)] + CMD_BASE)
                for s in range(ARG_NUM):
                    idx[b, G * (s + 1) + j, pl.ds(g8 * 16, 16)] = (
                        rawa[b, s, pl.ds(t0, 16)] + (s * ROW_STRIDE + 1))
            return carry
        lax.fori_loop(0, 8, body, 0)

    def zero_acc(b):
        z = jnp.zeros((16,), jnp.float32)

        def body(r, carry):
            for j in range(D // 16):
                acc[b, r, pl.ds(j * 16, 16)] = z
            return carry
        lax.fori_loop(0, CHUNK, body, 0)

    def fire_gathers(b):
        for k in range(17):
            for j in range(G):
                pltpu.async_copy(table.at[idx.at[b, G * k + j]],
                                 acc.at[b, pl.ds(j * 128, 128), :],
                                 sg[b], add=True)

    def drain_gathers(b):
        for _ in range(17 * G):
            pltpu.make_async_copy(table.at[idx.at[b, 0]],
                                  acc.at[b, pl.ds(0, 128), :], sg[b]).wait()

    def fire_out(b, base):
        pltpu.async_copy(acc.at[b], out.at[pl.ds(base, CHUNK), :], so[b])

    def wait_out(b):
        pltpu.make_async_copy(acc.at[b], out.at[pl.ds(0, CHUNK), :],
                              so[b]).wait()

    def chunk(b, base, first=False, second=False, fire_next=True):
        wait_raw(b, base)
        adjust(b)
        if fire_next:
            fire_raw(1 - b, base + CHUNK)
        if not (first or second):
            wait_out(b)
        zero_acc(b)
        fire_gathers(b)
        if not first:
            drain_gathers(1 - b)
            fire_out(1 - b, base - CHUNK)

    fire_raw(0, base0)
    chunk(0, base0, first=True)
    chunk(1, base0 + CHUNK, second=True)

    def pair(p, carry):
        base = base0 + 2 * p * CHUNK
        chunk(0, base)
        chunk(1, base + CHUNK)
        return carry
    lax.fori_loop(1, NCHUNK // 2 - 1, pair, 0)

    baseL = base0 + (NCHUNK - 2) * CHUNK
    chunk(0, baseL)
    chunk(1, baseL + CHUNK, fire_next=False)
    drain_gathers(1)
    fire_out(1, baseL + CHUNK)
    wait_out(0)
    wait_out(1)


_sc_lookup = functools.partial(
    pl.kernel,
    out_type=jax.ShapeDtypeStruct((B, D), jnp.float32),
    mesh=plsc.VectorSubcoreMesh(core_axis_name="c", subcore_axis_name="s"),
    scratch_types=[
        pltpu.VMEM((2, CHUNK), jnp.int32),
        pltpu.VMEM((2, ARG_NUM, CHUNK), jnp.int32),
        pltpu.VMEM((2, 17 * G, 128), jnp.int32),
        pltpu.VMEM((2, CHUNK, D), jnp.float32),
        pltpu.VMEM_SHARED((TBL_ROWS, D), jnp.float32),
        pltpu.SemaphoreType.DMA,
        pltpu.SemaphoreType.DMA,
        pltpu.SemaphoreType.DMA,
        pltpu.SemaphoreType.DMA,
        pltpu.SemaphoreType.DMA,
        pltpu.SemaphoreType.DMA,
        pltpu.SemaphoreType.DMA,
        pltpu.SemaphoreType.DMA,
    ],
)(_sc_body)


def kernel(commands, args, cmd_table, arg_table, W, b):
    flat_table = _fold(cmd_table, arg_table, W, b.reshape(1, D))
    argsT = args.reshape(B, ARG_NUM).T
    out = _sc_lookup(flat_table, commands.reshape(B), argsT)
    return out.reshape(N, S, D)


# speedup vs baseline: 17.9412x; 1.0461x over previous
import functools

import jax
import jax.numpy as jnp
from jax import lax
from jax.experimental import pallas as pl
from jax.experimental.pallas import tpu as pltpu
from jax.experimental.pallas import tpu_sc as plsc

N, S, ARG_NUM = 1024, 64, 16
B = N * S
D = 128
E = 64
ROW_STRIDE = 264
MRG_ROWS = 6 * ROW_STRIDE
TBL_ROWS = MRG_ROWS + 15 * ROW_STRIDE
NC, NS = 2, 16
NW = NC * NS
CHUNK = 256
G = CHUNK // 128
NCHUNK = (B // NW) // CHUNK
TOK_PER_W = B // NW


def _fold_body(cmd_ref, arg_ref, w_ref, b_ref, out_ref):
    a = arg_ref[...]
    t0 = jnp.dot(a, w_ref[pl.ds(0, E), :],
                 preferred_element_type=jnp.float32)
    cb = cmd_ref[...] + b_ref[...]
    for c in range(6):
        out_ref[pl.ds(c * ROW_STRIDE, 257), :] = t0 + cb[c:c + 1, :]
    for k in range(1, ARG_NUM):
        wk = w_ref[pl.ds(k * E, E), :]
        out_ref[pl.ds(MRG_ROWS + (k - 1) * ROW_STRIDE, 257), :] = jnp.dot(
            a, wk, preferred_element_type=jnp.float32)


def _fold(cmd_table, arg_table, W, b2):
    return pl.pallas_call(
        _fold_body,
        out_shape=jax.ShapeDtypeStruct((TBL_ROWS, D), jnp.float32),
    )(cmd_table, arg_table, W, b2)


def _sc_body(table_hbm, cmdf, argsT, out, rawc, rawa, idx, acc, table,
             sg0, sg1, sr0, sr1, sra0, sra1, so0, so1):
    sg = [sg0, sg1]
    sr = [sr0, sr1]
    sra = [sra0, sra1]
    so = [so0, so1]
    sid = lax.axis_index("s")
    wid = sid * NC + lax.axis_index("c")
    base0 = wid * TOK_PER_W

    @pl.when(sid == 0)
    def _stage():
        pltpu.sync_copy(table_hbm, table)
    plsc.subcore_barrier()

    def wait_raw(b, base):
        pltpu.make_async_copy(cmdf.at[pl.ds(base, CHUNK)],
                              rawc.at[b], sr[b]).wait()
        pltpu.make_async_copy(argsT.at[:, pl.ds(base, CHUNK)],
                              rawa.at[b], sra[b]).wait()

    def fire_raw(b, base):
        pltpu.async_copy(cmdf.at[pl.ds(base, CHUNK)], rawc.at[b], sr[b])
        pltpu.async_copy(argsT.at[:, pl.ds(base, CHUNK)], rawa.at[b], sra[b])

    def adjust(b):
        def body(g8, carry):
            for j in range(G):
                t0 = j * 128 + g8 * 16
                idx[b, j, pl.ds(g8 * 16, 16)] = (
                    rawc[b, pl.ds(t0, 16)] * ROW_STRIDE
                    + rawa[b, 0, pl.ds(t0, 16)] + 1)
                for s in range(1, ARG_NUM):
                    idx[b, G * s + j, pl.ds(g8 * 16, 16)] = (
                        rawa[b, s, pl.ds(t0, 16)]
                        + (MRG_ROWS + (s - 1) * ROW_STRIDE + 1))
            return carry
        lax.fori_loop(0, 8, body, 0)

    def zero_acc(b):
        z = jnp.zeros((16,), jnp.float32)

        def body(r, carry):
            for j in range(D // 16):
                acc[b, r, pl.ds(j * 16, 16)] = z
            return carry
        lax.fori_loop(0, CHUNK, body, 0)

    def fire_gathers(b):
        for k in range(ARG_NUM):
            for j in range(G):
                pltpu.async_copy(table.at[idx.at[b, G * k + j]],
                                 acc.at[b, pl.ds(j * 128, 128), :],
                                 sg[b], add=True)

    def drain_gathers(b):
        for _ in range(ARG_NUM * G):
            pltpu.make_async_copy(table.at[idx.at[b, 0]],
                                  acc.at[b, pl.ds(0, 128), :], sg[b]).wait()

    def fire_out(b, base):
        pltpu.async_copy(acc.at[b], out.at[pl.ds(base, CHUNK), :], so[b])

    def wait_out(b):
        pltpu.make_async_copy(acc.at[b], out.at[pl.ds(0, CHUNK), :],
                              so[b]).wait()

    def chunk(b, base, first=False, second=False, fire_next=True):
        wait_raw(b, base)
        adjust(b)
        if fire_next:
            fire_raw(1 - b, base + CHUNK)
        if not (first or second):
            wait_out(b)
        zero_acc(b)
        fire_gathers(b)
        if not first:
            drain_gathers(1 - b)
            fire_out(1 - b, base - CHUNK)

    fire_raw(0, base0)
    chunk(0, base0, first=True)
    chunk(1, base0 + CHUNK, second=True)

    def pair(p, carry):
        base = base0 + 2 * p * CHUNK
        chunk(0, base)
        chunk(1, base + CHUNK)
        return carry
    lax.fori_loop(1, NCHUNK // 2 - 1, pair, 0)

    baseL = base0 + (NCHUNK - 2) * CHUNK
    chunk(0, baseL)
    chunk(1, baseL + CHUNK, fire_next=False)
    drain_gathers(1)
    fire_out(1, baseL + CHUNK)
    wait_out(0)
    wait_out(1)


_sc_lookup = functools.partial(
    pl.kernel,
    out_type=jax.ShapeDtypeStruct((B, D), jnp.float32),
    mesh=plsc.VectorSubcoreMesh(core_axis_name="c", subcore_axis_name="s"),
    scratch_types=[
        pltpu.VMEM((2, CHUNK), jnp.int32),
        pltpu.VMEM((2, ARG_NUM, CHUNK), jnp.int32),
        pltpu.VMEM((2, ARG_NUM * G, 128), jnp.int32),
        pltpu.VMEM((2, CHUNK, D), jnp.float32),
        pltpu.VMEM_SHARED((TBL_ROWS, D), jnp.float32),
        pltpu.SemaphoreType.DMA,
        pltpu.SemaphoreType.DMA,
        pltpu.SemaphoreType.DMA,
        pltpu.SemaphoreType.DMA,
        pltpu.SemaphoreType.DMA,
        pltpu.SemaphoreType.DMA,
        pltpu.SemaphoreType.DMA,
        pltpu.SemaphoreType.DMA,
    ],
)(_sc_body)


def kernel(commands, args, cmd_table, arg_table, W, b):
    flat_table = _fold(cmd_table, arg_table, W, b.reshape(1, D))
    argsT = args.reshape(B, ARG_NUM).T
    out = _sc_lookup(flat_table, commands.reshape(B), argsT)
    return out.reshape(N, S, D)
